# baseline (speedup 1.0000x reference)
import functools

import jax
import jax.numpy as jnp
from jax import lax
from jax.experimental import pallas as pl
from jax.experimental.pallas import tpu as pltpu
from jax.experimental.pallas import tpu_sc as plsc

_NC = 2
_NS = 16
_NW = _NC * _NS
_SEQ_PER_CHUNK = 4
_GATHER = 100


def kernel(X, word_table, pos_table):
    B, T = X.shape
    V, D = word_table.shape
    N = B * T
    seqs_per_w = B // _NW
    chunks_per_w = seqs_per_w // _SEQ_PER_CHUNK
    chunk_rows = _SEQ_PER_CHUNK * T
    gpc = chunk_rows // _GATHER
    irows = N // _GATHER

    x2d = X.reshape(irows, _GATHER)

    mesh = plsc.VectorSubcoreMesh(core_axis_name="c", subcore_axis_name="s")

    @functools.partial(
        pl.kernel,
        out_type=jax.ShapeDtypeStruct((B, T, D), jnp.float32),
        mesh=mesh,
        scratch_types=[
            pltpu.VMEM((gpc, _GATHER), jnp.int32),
            pltpu.VMEM((chunk_rows, D), jnp.float32),
            pltpu.VMEM((T, D), jnp.float32),
            pltpu.SemaphoreType.DMA,
        ],
        compiler_params=pltpu.CompilerParams(use_tc_tiling_on_sc=False),
    )
    def emb(x_hbm, tab_hbm, pos_hbm, out_hbm, idx_v, rows_v, pos_v, sem):
        wid = lax.axis_index("s") * _NC + lax.axis_index("c")
        seq_base = wid * seqs_per_w
        pltpu.sync_copy(pos_hbm, pos_v)

        def chunk_body(it, carry):
            seq0 = seq_base + it * _SEQ_PER_CHUNK
            irow0 = pl.multiple_of(
                (seq_base * T) // _GATHER + it * gpc, gpc
            )
            pltpu.sync_copy(x_hbm.at[pl.ds(irow0, gpc)], idx_v)
            cps = [
                pltpu.async_copy(
                    tab_hbm.at[idx_v.at[j]],
                    rows_v.at[pl.ds(j * _GATHER, _GATHER)],
                    sem,
                )
                for j in range(gpc)
            ]
            for cp in cps:
                cp.wait()

            def add_row(r, c2):
                for s in range(_SEQ_PER_CHUNK):
                    for c in range(D // 16):
                        sl = pl.ds(c * 16, 16)
                        rows_v[s * T + r, sl] = rows_v[s * T + r, sl] + pos_v[r, sl]
                return c2

            lax.fori_loop(0, T, add_row, 0)
            for s in range(_SEQ_PER_CHUNK):
                pltpu.sync_copy(
                    rows_v.at[pl.ds(s * T, T)], out_hbm.at[seq0 + s]
                )
            return carry

        lax.fori_loop(0, chunks_per_w, chunk_body, 0)

    return emb(x2d, word_table, pos_table)
